# Initial kernel scaffold; baseline (speedup 1.0000x reference)
import jax
import jax.numpy as jnp
from jax.experimental import pallas as pl


def kernel(x, edge_index, W1, b1, W2, b2):
    raise NotImplementedError("write your pallas kernel here")


# speedup vs baseline: 15.7114x; 15.7114x over previous
import functools

import jax
import jax.numpy as jnp
from jax import lax
from jax.experimental import pallas as pl
from jax.experimental.pallas import tpu as pltpu, tpu_sc as plsc

NP = 10240
BR = 1024
NC = 2
NS = 16


def _zero_1d(ref, nwords):
    z = jnp.zeros((16,), jnp.float32)

    def body(i, _):
        ref[pl.ds(i * 16, 16)] = z
        return 0

    lax.fori_loop(0, nwords // 16, body, 0, unroll=4)


def _deg_body(dst_hbm, deg_hbm, hist_v, idx_v, red_v, shared_h):
    c = lax.axis_index("c")
    s = lax.axis_index("s")
    _zero_1d(hist_v, NP)

    e_tile = 5000
    base = c * (NS * e_tile) + s * e_tile
    pltpu.sync_copy(dst_hbm.at[pl.ds(base, e_tile)], idx_v)

    ones = jnp.ones((16,), jnp.float32)

    def body(k, _):
        idx = idx_v[pl.ds(k * 16, 16)]
        plsc.addupdate_scatter(hist_v, [idx], ones)
        return 0

    lax.fori_loop(0, e_tile // 16, body, 0)
    tail_idx = idx_v[pl.ds(e_tile - 16, 16)]
    tail_mask = lax.iota(jnp.int32, 16) >= 8
    plsc.addupdate_scatter(hist_v, [tail_idx], ones, mask=tail_mask)

    pltpu.sync_copy(hist_v, shared_h.at[s])
    plsc.subcore_barrier()
    rows = NP // NS
    for r in range(NS):
        pltpu.sync_copy(shared_h.at[r, pl.ds(s * rows, rows)], red_v.at[r])

    lax.fori_loop(0, rows // 16, _make_sum(red_v, hist_v), 0)
    pltpu.sync_copy(hist_v.at[pl.ds(0, rows)], deg_hbm.at[c, pl.ds(s * rows, rows)])


def _make_sum(red_v, out_v):
    def rbody(k, _):
        acc = red_v[0, pl.ds(k * 16, 16)]
        for r in range(1, NS):
            acc = acc + red_v[r, pl.ds(k * 16, 16)]
        out_v[pl.ds(k * 16, 16)] = acc
        return 0

    return rbody


def _deg_kernel(dst):
    mesh = plsc.VectorSubcoreMesh(core_axis_name="c", subcore_axis_name="s")
    f = pl.kernel(
        _deg_body,
        out_type=jax.ShapeDtypeStruct((NC, NP), jnp.float32),
        mesh=mesh,
        compiler_params=pltpu.CompilerParams(needs_layout_passes=False, use_tc_tiling_on_sc=False),
        scratch_types=[
            pltpu.VMEM((NP,), jnp.float32),
            pltpu.VMEM((5000,), jnp.int32),
            pltpu.VMEM((NS, NP // NS), jnp.float32),
            pltpu.VMEM_SHARED((NS, NP), jnp.float32),
        ],
    )
    return f(dst)


def _agg256_body(y0, y1, src, dst, zeros_hbm, t0, t1, acc_sh, idx_s, idx_d,
                 rows_v, idx_s_t, idx_d_t, rows_t, zrow_v, sem):
    c = lax.axis_index("c")
    s = lax.axis_index("s")
    rows = NP // NS

    def run(y_hbm, t_hbm):
        pltpu.sync_copy(zeros_hbm, zrow_v)
        for k in range(rows // 128):
            pltpu.sync_copy(zrow_v, acc_sh.at[pl.ds(s * rows + k * 128, 128)])
        plsc.subcore_barrier()

        e_tile = 10000
        base = s * e_tile

        def body(j, _):
            off = base + j * 128
            pltpu.sync_copy(src.at[pl.ds(off, 128)], idx_s)
            pltpu.sync_copy(dst.at[pl.ds(off, 128)], idx_d)
            pltpu.async_copy(y_hbm.at[idx_s], rows_v, sem).wait()
            pltpu.sync_copy(rows_v, acc_sh.at[idx_d], add=True)
            return 0

        lax.fori_loop(0, e_tile // 128, body, 0)
        off = base + (e_tile // 128) * 128
        pltpu.sync_copy(src.at[pl.ds(off, 16)], idx_s_t)
        pltpu.sync_copy(dst.at[pl.ds(off, 16)], idx_d_t)
        pltpu.async_copy(y_hbm.at[idx_s_t], rows_t, sem).wait()
        pltpu.sync_copy(rows_t, acc_sh.at[idx_d_t], add=True)

        plsc.subcore_barrier()
        for k in range(rows // 128):
            r0 = s * rows + k * 128
            pltpu.sync_copy(acc_sh.at[pl.ds(r0, 128)], t_hbm.at[pl.ds(r0, 128)])

    @pl.when(c == 0)
    def _():
        run(y0, t0)

    @pl.when(c == 1)
    def _():
        run(y1, t1)


def _agg256_kernel(y0, y1, src, dst, zeros128):
    mesh = plsc.VectorSubcoreMesh(core_axis_name="c", subcore_axis_name="s")
    f = pl.kernel(
        _agg256_body,
        out_type=[
            jax.ShapeDtypeStruct((NP, 128), jnp.float32),
            jax.ShapeDtypeStruct((NP, 128), jnp.float32),
        ],
        mesh=mesh,
        compiler_params=pltpu.CompilerParams(needs_layout_passes=False, use_tc_tiling_on_sc=False),
        scratch_types=[
            pltpu.VMEM_SHARED((NP, 128), jnp.float32),
            pltpu.VMEM((128,), jnp.int32),
            pltpu.VMEM((128,), jnp.int32),
            pltpu.VMEM((128, 128), jnp.float32),
            pltpu.VMEM((16,), jnp.int32),
            pltpu.VMEM((16,), jnp.int32),
            pltpu.VMEM((16, 128), jnp.float32),
            pltpu.VMEM((128, 128), jnp.float32),
            pltpu.SemaphoreType.DMA,
        ],
    )
    return f(y0, y1, src, dst, zeros128)


def _agg16_body(y2, src, dst, zeros_hbm, u_hbm, acc_sh, idx_s, idx_d, rows_v,
                idx_s_t, idx_d_t, rows_t, zrow_v, sem):
    c = lax.axis_index("c")
    s = lax.axis_index("s")
    rows = NP // NS

    pltpu.sync_copy(zeros_hbm, zrow_v)
    for k in range(rows // 128):
        pltpu.sync_copy(zrow_v, acc_sh.at[pl.ds(s * rows + k * 128, 128)])
    plsc.subcore_barrier()

    e_tile = 5000
    base = c * (NS * e_tile) + s * e_tile

    def body(j, _):
        off = base + j * 128
        pltpu.sync_copy(src.at[pl.ds(off, 128)], idx_s)
        pltpu.sync_copy(dst.at[pl.ds(off, 128)], idx_d)
        pltpu.async_copy(y2.at[idx_s], rows_v, sem).wait()
        pltpu.sync_copy(rows_v, acc_sh.at[idx_d], add=True)
        return 0

    lax.fori_loop(0, e_tile // 128, body, 0)
    off = base + (e_tile // 128) * 128
    pltpu.sync_copy(src.at[pl.ds(off, 8)], idx_s_t)
    pltpu.sync_copy(dst.at[pl.ds(off, 8)], idx_d_t)
    pltpu.async_copy(y2.at[idx_s_t], rows_t, sem).wait()
    pltpu.sync_copy(rows_t, acc_sh.at[idx_d_t], add=True)

    plsc.subcore_barrier()
    for k in range(rows // 128):
        r0 = s * rows + k * 128
        pltpu.sync_copy(acc_sh.at[pl.ds(r0, 128)], u_hbm.at[c, pl.ds(r0, 128)])


def _agg16_kernel(y2, src, dst, zeros16):
    mesh = plsc.VectorSubcoreMesh(core_axis_name="c", subcore_axis_name="s")
    f = pl.kernel(
        _agg16_body,
        out_type=jax.ShapeDtypeStruct((NC, NP, 16), jnp.float32),
        mesh=mesh,
        compiler_params=pltpu.CompilerParams(needs_layout_passes=False, use_tc_tiling_on_sc=False),
        scratch_types=[
            pltpu.VMEM_SHARED((NP, 16), jnp.float32),
            pltpu.VMEM((128,), jnp.int32),
            pltpu.VMEM((128,), jnp.int32),
            pltpu.VMEM((128, 16), jnp.float32),
            pltpu.VMEM((8,), jnp.int32),
            pltpu.VMEM((8,), jnp.int32),
            pltpu.VMEM((8, 16), jnp.float32),
            pltpu.VMEM((128, 16), jnp.float32),
            pltpu.SemaphoreType.DMA,
        ],
    )
    return f(y2, src, dst, zeros16)


def _tca_body(deg2_ref, x_ref, dinv_ref, y0_ref, y1_ref):
    d = deg2_ref[...]
    dinv = lax.rsqrt(d[:, 0:1] + d[:, 1:2] + 1.0)
    y = x_ref[...] * dinv
    dinv_ref[...] = jnp.broadcast_to(dinv, (BR, 8))
    y0_ref[...] = y[:, :128]
    y1_ref[...] = y[:, 128:]


def _tca(deg2, x_p):
    return pl.pallas_call(
        _tca_body,
        grid=(NP // BR,),
        in_specs=[
            pl.BlockSpec((BR, 2), lambda i: (i, 0)),
            pl.BlockSpec((BR, 256), lambda i: (i, 0)),
        ],
        out_specs=[
            pl.BlockSpec((BR, 8), lambda i: (i, 0)),
            pl.BlockSpec((BR, 128), lambda i: (i, 0)),
            pl.BlockSpec((BR, 128), lambda i: (i, 0)),
        ],
        out_shape=[
            jax.ShapeDtypeStruct((NP, 8), jnp.float32),
            jax.ShapeDtypeStruct((NP, 128), jnp.float32),
            jax.ShapeDtypeStruct((NP, 128), jnp.float32),
        ],
    )(deg2, x_p)


def _tcb_body(t0_ref, t1_ref, y0_ref, y1_ref, dinv_ref, w1_ref, b1_ref,
              w2_ref, y2_ref):
    dinv = dinv_ref[...][:, 0:1]
    a0 = (t0_ref[...] + y0_ref[...]) * dinv
    a1 = (t1_ref[...] + y1_ref[...]) * dinv
    a = jnp.concatenate([a0, a1], axis=1)
    h = jnp.dot(a, w1_ref[...], preferred_element_type=jnp.float32)
    h = jnp.maximum(h + b1_ref[...], 0.0)
    z = jnp.dot(h, w2_ref[...], preferred_element_type=jnp.float32)
    y2_ref[...] = z * dinv


def _tcb(t0, t1, y0, y1, dinv, W1, b1r, W2):
    return pl.pallas_call(
        _tcb_body,
        grid=(NP // BR,),
        in_specs=[
            pl.BlockSpec((BR, 128), lambda i: (i, 0)),
            pl.BlockSpec((BR, 128), lambda i: (i, 0)),
            pl.BlockSpec((BR, 128), lambda i: (i, 0)),
            pl.BlockSpec((BR, 128), lambda i: (i, 0)),
            pl.BlockSpec((BR, 8), lambda i: (i, 0)),
            pl.BlockSpec((256, 512), lambda i: (0, 0)),
            pl.BlockSpec((1, 512), lambda i: (0, 0)),
            pl.BlockSpec((512, 16), lambda i: (0, 0)),
        ],
        out_specs=pl.BlockSpec((BR, 16), lambda i: (i, 0)),
        out_shape=jax.ShapeDtypeStruct((NP, 16), jnp.float32),
    )(t0, t1, y0, y1, dinv, W1, b1r, W2)


def _tcc_body(u_ref, y2_ref, dinv_ref, b2_ref, out_ref):
    u = u_ref[...]
    o = (u[0] + u[1] + y2_ref[...]) * dinv_ref[...][:, 0:1] + b2_ref[...]
    m = jnp.max(o, axis=1, keepdims=True)
    l = o - m
    out_ref[...] = l - jnp.log(jnp.sum(jnp.exp(l), axis=1, keepdims=True))


def _tcc(u, y2, dinv, b2r, n_out):
    return pl.pallas_call(
        _tcc_body,
        grid=(NP // BR,),
        in_specs=[
            pl.BlockSpec((NC, BR, 16), lambda i: (0, i, 0)),
            pl.BlockSpec((BR, 16), lambda i: (i, 0)),
            pl.BlockSpec((BR, 8), lambda i: (i, 0)),
            pl.BlockSpec((1, 16), lambda i: (0, 0)),
        ],
        out_specs=pl.BlockSpec((BR, 16), lambda i: (i, 0)),
        out_shape=jax.ShapeDtypeStruct((n_out, 16), jnp.float32),
    )(u, y2, dinv, b2r)


@jax.jit
def kernel(x, edge_index, W1, b1, W2, b2):
    n = x.shape[0]
    src = edge_index[0]
    dst = edge_index[1]
    x_p = jnp.zeros((NP, x.shape[1]), jnp.float32).at[:n].set(x)
    zeros128 = jnp.zeros((128, 128), jnp.float32)
    zeros16 = jnp.zeros((128, 16), jnp.float32)

    deg_c = _deg_kernel(dst)
    deg2 = deg_c.T
    dinv, y0, y1 = _tca(deg2, x_p)
    t0, t1 = _agg256_kernel(y0, y1, src, dst, zeros128)
    y2 = _tcb(t0, t1, y0, y1, dinv, W1, b1.reshape(1, -1), W2)
    u = _agg16_kernel(y2, src, dst, zeros16)
    return _tcc(u, y2, dinv, b2.reshape(1, -1), n)


# speedup vs baseline: 17.9346x; 1.1415x over previous
import functools

import jax
import jax.numpy as jnp
from jax import lax
from jax.experimental import pallas as pl
from jax.experimental.pallas import tpu as pltpu, tpu_sc as plsc

NP = 10240
BR = 1024
NC = 2
NS = 16


def _zero_1d(ref, nwords):
    z = jnp.zeros((16,), jnp.float32)

    def body(i, _):
        ref[pl.ds(i * 16, 16)] = z
        return 0

    lax.fori_loop(0, nwords // 16, body, 0, unroll=4)


def _deg_body(dst_hbm, deg_hbm, hist_v, idx_v, red_v, shared_h):
    c = lax.axis_index("c")
    s = lax.axis_index("s")
    _zero_1d(hist_v, NP)

    e_tile = 5000
    base = c * (NS * e_tile) + s * e_tile
    pltpu.sync_copy(dst_hbm.at[pl.ds(base, e_tile)], idx_v)

    ones = jnp.ones((16,), jnp.float32)

    def body(k, _):
        idx = idx_v[pl.ds(k * 16, 16)]
        plsc.addupdate_scatter(hist_v, [idx], ones)
        return 0

    lax.fori_loop(0, e_tile // 16, body, 0)
    tail_idx = idx_v[pl.ds(e_tile - 16, 16)]
    tail_mask = lax.iota(jnp.int32, 16) >= 8
    plsc.addupdate_scatter(hist_v, [tail_idx], ones, mask=tail_mask)

    pltpu.sync_copy(hist_v, shared_h.at[s])
    plsc.subcore_barrier()
    rows = NP // NS
    for r in range(NS):
        pltpu.sync_copy(shared_h.at[r, pl.ds(s * rows, rows)], red_v.at[r])

    lax.fori_loop(0, rows // 16, _make_sum(red_v, hist_v), 0)
    pltpu.sync_copy(hist_v.at[pl.ds(0, rows)], deg_hbm.at[c, pl.ds(s * rows, rows)])


def _make_sum(red_v, out_v):
    def rbody(k, _):
        acc = red_v[0, pl.ds(k * 16, 16)]
        for r in range(1, NS):
            acc = acc + red_v[r, pl.ds(k * 16, 16)]
        out_v[pl.ds(k * 16, 16)] = acc
        return 0

    return rbody


def _deg_kernel(dst):
    mesh = plsc.VectorSubcoreMesh(core_axis_name="c", subcore_axis_name="s")
    f = pl.kernel(
        _deg_body,
        out_type=jax.ShapeDtypeStruct((NC, NP), jnp.float32),
        mesh=mesh,
        compiler_params=pltpu.CompilerParams(needs_layout_passes=False, use_tc_tiling_on_sc=False),
        scratch_types=[
            pltpu.VMEM((NP,), jnp.float32),
            pltpu.VMEM((5000,), jnp.int32),
            pltpu.VMEM((NS, NP // NS), jnp.float32),
            pltpu.VMEM_SHARED((NS, NP), jnp.float32),
        ],
    )
    return f(dst)


def _edge_pipeline(src, dst, y_hbm, acc_sh, base, nchunks, b0, b1):
    idx_s0, idx_d0, rows0, gs0, ss0 = b0
    idx_s1, idx_d1, rows1, gs1, ss1 = b1
    CH = rows0.shape[0]

    def load(idx_s, idx_d, off):
        pltpu.sync_copy(src.at[pl.ds(off, CH)], idx_s)
        pltpu.sync_copy(dst.at[pl.ds(off, CH)], idx_d)

    def gstart(idx_s, rows, gs):
        pltpu.async_copy(y_hbm.at[idx_s], rows, gs)

    def gwait(idx_s, rows, gs):
        pltpu.make_async_copy(y_hbm.at[idx_s], rows, gs).wait()

    def sstart(idx_d, rows, ss):
        pltpu.async_copy(rows, acc_sh.at[idx_d], ss, add=True)

    def swait(idx_d, rows, ss):
        pltpu.make_async_copy(rows, acc_sh.at[idx_d], ss).wait()

    load(idx_s0, idx_d0, base)
    gstart(idx_s0, rows0, gs0)
    gwait(idx_s0, rows0, gs0)
    sstart(idx_d0, rows0, ss0)
    load(idx_s1, idx_d1, base + CH)
    gstart(idx_s1, rows1, gs1)
    gwait(idx_s1, rows1, gs1)
    sstart(idx_d1, rows1, ss1)
    swait(idx_d0, rows0, ss0)
    load(idx_s0, idx_d0, base + 2 * CH)
    gstart(idx_s0, rows0, gs0)

    npairs = (nchunks - 2) // 2

    def pair(k, _):
        m0 = 2 * k + 2
        gwait(idx_s0, rows0, gs0)
        sstart(idx_d0, rows0, ss0)
        swait(idx_d1, rows1, ss1)
        load(idx_s1, idx_d1, base + (m0 + 1) * CH)
        gstart(idx_s1, rows1, gs1)
        gwait(idx_s1, rows1, gs1)
        sstart(idx_d1, rows1, ss1)
        swait(idx_d0, rows0, ss0)

        @pl.when(m0 + 2 < nchunks)
        def _():
            load(idx_s0, idx_d0, base + (m0 + 2) * CH)
            gstart(idx_s0, rows0, gs0)

        return 0

    lax.fori_loop(0, npairs, pair, 0)
    if nchunks % 2 == 0:
        swait(idx_d1, rows1, ss1)
    else:
        gwait(idx_s0, rows0, gs0)
        sstart(idx_d0, rows0, ss0)
        swait(idx_d1, rows1, ss1)
        swait(idx_d0, rows0, ss0)


def _agg256_body(y0, y1, src, dst, zeros_hbm, t0, t1, acc_sh,
                 idx_s0, idx_d0, rows0, idx_s1, idx_d1, rows1,
                 idx_s_t, idx_d_t, rows_t, gs0, ss0, gs1, ss1, sem):
    c = lax.axis_index("c")
    s = lax.axis_index("s")
    rows = NP // NS

    def run(y_hbm, t_hbm):
        pltpu.sync_copy(zeros_hbm, rows0)
        for k in range(rows // 128):
            pltpu.sync_copy(rows0, acc_sh.at[pl.ds(s * rows + k * 128, 128)])
        plsc.subcore_barrier()

        e_tile = 10000
        base = s * e_tile
        _edge_pipeline(src, dst, y_hbm, acc_sh, base, e_tile // 128,
                       (idx_s0, idx_d0, rows0, gs0, ss0),
                       (idx_s1, idx_d1, rows1, gs1, ss1))
        off = base + (e_tile // 128) * 128
        pltpu.sync_copy(src.at[pl.ds(off, 16)], idx_s_t)
        pltpu.sync_copy(dst.at[pl.ds(off, 16)], idx_d_t)
        pltpu.async_copy(y_hbm.at[idx_s_t], rows_t, sem).wait()
        pltpu.sync_copy(rows_t, acc_sh.at[idx_d_t], add=True)

        plsc.subcore_barrier()
        for k in range(rows // 128):
            r0 = s * rows + k * 128
            pltpu.sync_copy(acc_sh.at[pl.ds(r0, 128)], t_hbm.at[pl.ds(r0, 128)])

    @pl.when(c == 0)
    def _():
        run(y0, t0)

    @pl.when(c == 1)
    def _():
        run(y1, t1)


def _agg256_kernel(y0, y1, src, dst, zeros128):
    mesh = plsc.VectorSubcoreMesh(core_axis_name="c", subcore_axis_name="s")
    f = pl.kernel(
        _agg256_body,
        out_type=[
            jax.ShapeDtypeStruct((NP, 128), jnp.float32),
            jax.ShapeDtypeStruct((NP, 128), jnp.float32),
        ],
        mesh=mesh,
        compiler_params=pltpu.CompilerParams(needs_layout_passes=False, use_tc_tiling_on_sc=False),
        scratch_types=[
            pltpu.VMEM_SHARED((NP, 128), jnp.float32),
            pltpu.VMEM((128,), jnp.int32),
            pltpu.VMEM((128,), jnp.int32),
            pltpu.VMEM((128, 128), jnp.float32),
            pltpu.VMEM((128,), jnp.int32),
            pltpu.VMEM((128,), jnp.int32),
            pltpu.VMEM((128, 128), jnp.float32),
            pltpu.VMEM((16,), jnp.int32),
            pltpu.VMEM((16,), jnp.int32),
            pltpu.VMEM((16, 128), jnp.float32),
            pltpu.SemaphoreType.DMA,
            pltpu.SemaphoreType.DMA,
            pltpu.SemaphoreType.DMA,
            pltpu.SemaphoreType.DMA,
            pltpu.SemaphoreType.DMA,
        ],
    )
    return f(y0, y1, src, dst, zeros128)


def _agg16_body(y2, src, dst, zeros_hbm, u_hbm, acc_sh,
                idx_s0, idx_d0, rows0, idx_s1, idx_d1, rows1,
                idx_s_t, idx_d_t, rows_t, gs0, ss0, gs1, ss1, sem):
    c = lax.axis_index("c")
    s = lax.axis_index("s")
    rows = NP // NS

    pltpu.sync_copy(zeros_hbm, rows0)
    for k in range(rows // 128):
        pltpu.sync_copy(rows0, acc_sh.at[pl.ds(s * rows + k * 128, 128)])
    plsc.subcore_barrier()

    e_tile = 5000
    base = c * (NS * e_tile) + s * e_tile
    _edge_pipeline(src, dst, y2, acc_sh, base, e_tile // 128,
                   (idx_s0, idx_d0, rows0, gs0, ss0),
                   (idx_s1, idx_d1, rows1, gs1, ss1))
    off = base + (e_tile // 128) * 128
    pltpu.sync_copy(src.at[pl.ds(off, 8)], idx_s_t)
    pltpu.sync_copy(dst.at[pl.ds(off, 8)], idx_d_t)
    pltpu.async_copy(y2.at[idx_s_t], rows_t, sem).wait()
    pltpu.sync_copy(rows_t, acc_sh.at[idx_d_t], add=True)

    plsc.subcore_barrier()
    for k in range(rows // 128):
        r0 = s * rows + k * 128
        pltpu.sync_copy(acc_sh.at[pl.ds(r0, 128)], u_hbm.at[c, pl.ds(r0, 128)])


def _agg16_kernel(y2, src, dst, zeros16):
    mesh = plsc.VectorSubcoreMesh(core_axis_name="c", subcore_axis_name="s")
    f = pl.kernel(
        _agg16_body,
        out_type=jax.ShapeDtypeStruct((NC, NP, 16), jnp.float32),
        mesh=mesh,
        compiler_params=pltpu.CompilerParams(needs_layout_passes=False, use_tc_tiling_on_sc=False),
        scratch_types=[
            pltpu.VMEM_SHARED((NP, 16), jnp.float32),
            pltpu.VMEM((128,), jnp.int32),
            pltpu.VMEM((128,), jnp.int32),
            pltpu.VMEM((128, 16), jnp.float32),
            pltpu.VMEM((128,), jnp.int32),
            pltpu.VMEM((128,), jnp.int32),
            pltpu.VMEM((128, 16), jnp.float32),
            pltpu.VMEM((8,), jnp.int32),
            pltpu.VMEM((8,), jnp.int32),
            pltpu.VMEM((8, 16), jnp.float32),
            pltpu.SemaphoreType.DMA,
            pltpu.SemaphoreType.DMA,
            pltpu.SemaphoreType.DMA,
            pltpu.SemaphoreType.DMA,
            pltpu.SemaphoreType.DMA,
        ],
    )
    return f(y2, src, dst, zeros16)


def _tca_body(deg2_ref, x_ref, dinv_ref, y0_ref, y1_ref):
    d = deg2_ref[...]
    dinv = lax.rsqrt(d[:, 0:1] + d[:, 1:2] + 1.0)
    y = x_ref[...] * dinv
    dinv_ref[...] = jnp.broadcast_to(dinv, (BR, 8))
    y0_ref[...] = y[:, :128]
    y1_ref[...] = y[:, 128:]


def _tca(deg2, x_p):
    return pl.pallas_call(
        _tca_body,
        grid=(NP // BR,),
        in_specs=[
            pl.BlockSpec((BR, 2), lambda i: (i, 0)),
            pl.BlockSpec((BR, 256), lambda i: (i, 0)),
        ],
        out_specs=[
            pl.BlockSpec((BR, 8), lambda i: (i, 0)),
            pl.BlockSpec((BR, 128), lambda i: (i, 0)),
            pl.BlockSpec((BR, 128), lambda i: (i, 0)),
        ],
        out_shape=[
            jax.ShapeDtypeStruct((NP, 8), jnp.float32),
            jax.ShapeDtypeStruct((NP, 128), jnp.float32),
            jax.ShapeDtypeStruct((NP, 128), jnp.float32),
        ],
    )(deg2, x_p)


def _tcb_body(t0_ref, t1_ref, y0_ref, y1_ref, dinv_ref, w1_ref, b1_ref,
              w2_ref, y2_ref):
    dinv = dinv_ref[...][:, 0:1]
    a0 = (t0_ref[...] + y0_ref[...]) * dinv
    a1 = (t1_ref[...] + y1_ref[...]) * dinv
    a = jnp.concatenate([a0, a1], axis=1)
    h = jnp.dot(a, w1_ref[...], preferred_element_type=jnp.float32)
    h = jnp.maximum(h + b1_ref[...], 0.0)
    z = jnp.dot(h, w2_ref[...], preferred_element_type=jnp.float32)
    y2_ref[...] = z * dinv


def _tcb(t0, t1, y0, y1, dinv, W1, b1r, W2):
    return pl.pallas_call(
        _tcb_body,
        grid=(NP // BR,),
        in_specs=[
            pl.BlockSpec((BR, 128), lambda i: (i, 0)),
            pl.BlockSpec((BR, 128), lambda i: (i, 0)),
            pl.BlockSpec((BR, 128), lambda i: (i, 0)),
            pl.BlockSpec((BR, 128), lambda i: (i, 0)),
            pl.BlockSpec((BR, 8), lambda i: (i, 0)),
            pl.BlockSpec((256, 512), lambda i: (0, 0)),
            pl.BlockSpec((1, 512), lambda i: (0, 0)),
            pl.BlockSpec((512, 16), lambda i: (0, 0)),
        ],
        out_specs=pl.BlockSpec((BR, 16), lambda i: (i, 0)),
        out_shape=jax.ShapeDtypeStruct((NP, 16), jnp.float32),
    )(t0, t1, y0, y1, dinv, W1, b1r, W2)


def _tcc_body(u_ref, y2_ref, dinv_ref, b2_ref, out_ref):
    u = u_ref[...]
    o = (u[0] + u[1] + y2_ref[...]) * dinv_ref[...][:, 0:1] + b2_ref[...]
    m = jnp.max(o, axis=1, keepdims=True)
    l = o - m
    out_ref[...] = l - jnp.log(jnp.sum(jnp.exp(l), axis=1, keepdims=True))


def _tcc(u, y2, dinv, b2r, n_out):
    return pl.pallas_call(
        _tcc_body,
        grid=(NP // BR,),
        in_specs=[
            pl.BlockSpec((NC, BR, 16), lambda i: (0, i, 0)),
            pl.BlockSpec((BR, 16), lambda i: (i, 0)),
            pl.BlockSpec((BR, 8), lambda i: (i, 0)),
            pl.BlockSpec((1, 16), lambda i: (0, 0)),
        ],
        out_specs=pl.BlockSpec((BR, 16), lambda i: (i, 0)),
        out_shape=jax.ShapeDtypeStruct((n_out, 16), jnp.float32),
    )(u, y2, dinv, b2r)


@jax.jit
def kernel(x, edge_index, W1, b1, W2, b2):
    n = x.shape[0]
    src = edge_index[0]
    dst = edge_index[1]
    x_p = jnp.zeros((NP, x.shape[1]), jnp.float32).at[:n].set(x)
    zeros128 = jnp.zeros((128, 128), jnp.float32)
    zeros16 = jnp.zeros((128, 16), jnp.float32)

    deg_c = _deg_kernel(dst)
    deg2 = deg_c.T
    dinv, y0, y1 = _tca(deg2, x_p)
    t0, t1 = _agg256_kernel(y0, y1, src, dst, zeros128)
    y2 = _tcb(t0, t1, y0, y1, dinv, W1, b1.reshape(1, -1), W2)
    u = _agg16_kernel(y2, src, dst, zeros16)
    return _tcc(u, y2, dinv, b2.reshape(1, -1), n)


# speedup vs baseline: 18.4026x; 1.0261x over previous
import functools

import jax
import jax.numpy as jnp
from jax import lax
from jax.experimental import pallas as pl
from jax.experimental.pallas import tpu as pltpu, tpu_sc as plsc

NP = 10240
BR = 1024
NC = 2
NS = 16


def _zero_1d(ref, nwords):
    z = jnp.zeros((16,), jnp.float32)

    def body(i, _):
        ref[pl.ds(i * 16, 16)] = z
        return 0

    lax.fori_loop(0, nwords // 16, body, 0, unroll=4)


def _deg_body(ei_hbm, deg_hbm, hist_v, idx_v, red_v, shared_h):
    c = lax.axis_index("c")
    s = lax.axis_index("s")
    _zero_1d(hist_v, NP)

    e_tile = 5000
    base = c * (NS * e_tile) + s * e_tile
    pltpu.sync_copy(ei_hbm.at[1, pl.ds(base, e_tile)], idx_v)

    ones = jnp.ones((16,), jnp.float32)

    def body(k, _):
        idx = idx_v[pl.ds(k * 16, 16)]
        plsc.addupdate_scatter(hist_v, [idx], ones)
        return 0

    lax.fori_loop(0, e_tile // 16, body, 0)
    tail_idx = idx_v[pl.ds(e_tile - 16, 16)]
    tail_mask = lax.iota(jnp.int32, 16) >= 8
    plsc.addupdate_scatter(hist_v, [tail_idx], ones, mask=tail_mask)

    pltpu.sync_copy(hist_v, shared_h.at[s])
    plsc.subcore_barrier()
    rows = NP // NS
    for r in range(NS):
        pltpu.sync_copy(shared_h.at[r, pl.ds(s * rows, rows)], red_v.at[r])

    lax.fori_loop(0, rows // 16, _make_sum(red_v, hist_v), 0)
    pltpu.sync_copy(hist_v.at[pl.ds(0, rows)], deg_hbm.at[c, pl.ds(s * rows, rows)])


def _make_sum(red_v, out_v):
    def rbody(k, _):
        acc = red_v[0, pl.ds(k * 16, 16)]
        for r in range(1, NS):
            acc = acc + red_v[r, pl.ds(k * 16, 16)]
        out_v[pl.ds(k * 16, 16)] = acc
        return 0

    return rbody


def _deg_kernel(ei):
    mesh = plsc.VectorSubcoreMesh(core_axis_name="c", subcore_axis_name="s")
    f = pl.kernel(
        _deg_body,
        out_type=jax.ShapeDtypeStruct((NC, NP), jnp.float32),
        mesh=mesh,
        compiler_params=pltpu.CompilerParams(needs_layout_passes=False, use_tc_tiling_on_sc=False),
        scratch_types=[
            pltpu.VMEM((NP,), jnp.float32),
            pltpu.VMEM((5000,), jnp.int32),
            pltpu.VMEM((NS, NP // NS), jnp.float32),
            pltpu.VMEM_SHARED((NS, NP), jnp.float32),
        ],
    )
    return f(ei)


def _edge_pipeline(ei, y_hbm, acc_sh, base, nchunks, bufs):
    assert nchunks % 3 == 0
    CH = bufs[0][2].shape[0]

    def load(b, off):
        pltpu.sync_copy(ei.at[0, pl.ds(off, CH)], b[0])
        pltpu.sync_copy(ei.at[1, pl.ds(off, CH)], b[1])

    def gstart(b):
        pltpu.async_copy(y_hbm.at[b[0]], b[2], b[3])

    def gwait(b):
        pltpu.make_async_copy(y_hbm.at[b[0]], b[2], b[3]).wait()

    def sstart(b):
        pltpu.async_copy(b[2], acc_sh.at[b[1]], b[4], add=True)

    def swait(b):
        pltpu.make_async_copy(b[2], acc_sh.at[b[1]], b[4]).wait()

    load(bufs[0], base)
    gstart(bufs[0])
    load(bufs[1], base + CH)
    gstart(bufs[1])

    def body(k, _):
        for j in range(3):
            b = bufs[j]
            p = bufs[(j + 2) % 3]
            gwait(b)
            sstart(b)
            if j == 0:
                @pl.when(k > 0)
                def _():
                    swait(p)
            else:
                swait(p)
            if j == 0:
                load(p, base + (3 * k + j + 2) * CH)
                gstart(p)
            else:
                @pl.when(3 * k + j + 2 < nchunks)
                def _():
                    load(p, base + (3 * k + j + 2) * CH)
                    gstart(p)
        return 0

    lax.fori_loop(0, nchunks // 3, body, 0)
    swait(bufs[2])


def _agg256_body(y0, y1, ei, zeros_hbm, t0, t1, acc_sh,
                 idx_s0, idx_d0, rows0, idx_s1, idx_d1, rows1,
                 idx_s2, idx_d2, rows2,
                 idx_s_t, idx_d_t, rows_t, gs0, ss0, gs1, ss1, gs2, ss2, sem):
    c = lax.axis_index("c")
    s = lax.axis_index("s")
    rows = NP // NS

    def run(y_hbm, t_hbm):
        for k in range(rows // 128):
            pltpu.sync_copy(zeros_hbm, acc_sh.at[pl.ds(s * rows + k * 128, 128)])
        plsc.subcore_barrier()

        e_tile = 10000
        base = s * e_tile
        _edge_pipeline(ei, y_hbm, acc_sh, base, e_tile // 64,
                       ((idx_s0, idx_d0, rows0, gs0, ss0),
                        (idx_s1, idx_d1, rows1, gs1, ss1),
                        (idx_s2, idx_d2, rows2, gs2, ss2)))
        off = base + (e_tile // 64) * 64
        pltpu.sync_copy(ei.at[0, pl.ds(off, 16)], idx_s_t)
        pltpu.sync_copy(ei.at[1, pl.ds(off, 16)], idx_d_t)
        pltpu.async_copy(y_hbm.at[idx_s_t], rows_t, sem).wait()
        pltpu.sync_copy(rows_t, acc_sh.at[idx_d_t], add=True)

        plsc.subcore_barrier()
        for k in range(rows // 128):
            r0 = s * rows + k * 128
            pltpu.sync_copy(acc_sh.at[pl.ds(r0, 128)], t_hbm.at[pl.ds(r0, 128)])

    @pl.when(c == 0)
    def _():
        run(y0, t0)

    @pl.when(c == 1)
    def _():
        run(y1, t1)


def _agg256_kernel(y0, y1, ei, zeros128):
    mesh = plsc.VectorSubcoreMesh(core_axis_name="c", subcore_axis_name="s")
    f = pl.kernel(
        _agg256_body,
        out_type=[
            jax.ShapeDtypeStruct((NP, 128), jnp.float32),
            jax.ShapeDtypeStruct((NP, 128), jnp.float32),
        ],
        mesh=mesh,
        compiler_params=pltpu.CompilerParams(needs_layout_passes=False, use_tc_tiling_on_sc=False),
        scratch_types=[
            pltpu.VMEM_SHARED((NP, 128), jnp.float32),
            pltpu.VMEM((64,), jnp.int32),
            pltpu.VMEM((64,), jnp.int32),
            pltpu.VMEM((64, 128), jnp.float32),
            pltpu.VMEM((64,), jnp.int32),
            pltpu.VMEM((64,), jnp.int32),
            pltpu.VMEM((64, 128), jnp.float32),
            pltpu.VMEM((64,), jnp.int32),
            pltpu.VMEM((64,), jnp.int32),
            pltpu.VMEM((64, 128), jnp.float32),
            pltpu.VMEM((16,), jnp.int32),
            pltpu.VMEM((16,), jnp.int32),
            pltpu.VMEM((16, 128), jnp.float32),
            pltpu.SemaphoreType.DMA,
            pltpu.SemaphoreType.DMA,
            pltpu.SemaphoreType.DMA,
            pltpu.SemaphoreType.DMA,
            pltpu.SemaphoreType.DMA,
            pltpu.SemaphoreType.DMA,
            pltpu.SemaphoreType.DMA,
        ],
    )
    return f(y0, y1, ei, zeros128)


def _agg16_body(y2, ei, zeros_hbm, u_hbm, acc_sh,
                idx_s0, idx_d0, rows0, idx_s1, idx_d1, rows1,
                idx_s2, idx_d2, rows2,
                idx_s_t, idx_d_t, rows_t, gs0, ss0, gs1, ss1, gs2, ss2, sem):
    c = lax.axis_index("c")
    s = lax.axis_index("s")
    rows = NP // NS

    for k in range(rows // 64):
        pltpu.sync_copy(zeros_hbm, acc_sh.at[pl.ds(s * rows + k * 64, 64)])
    plsc.subcore_barrier()

    e_tile = 5000
    base = c * (NS * e_tile) + s * e_tile
    _edge_pipeline(ei, y2, acc_sh, base, e_tile // 64,
                   ((idx_s0, idx_d0, rows0, gs0, ss0),
                    (idx_s1, idx_d1, rows1, gs1, ss1),
                    (idx_s2, idx_d2, rows2, gs2, ss2)))
    off = base + (e_tile // 64) * 64
    pltpu.sync_copy(ei.at[0, pl.ds(off, 8)], idx_s_t)
    pltpu.sync_copy(ei.at[1, pl.ds(off, 8)], idx_d_t)
    pltpu.async_copy(y2.at[idx_s_t], rows_t, sem).wait()
    pltpu.sync_copy(rows_t, acc_sh.at[idx_d_t], add=True)

    plsc.subcore_barrier()
    for k in range(rows // 64):
        r0 = s * rows + k * 64
        pltpu.sync_copy(acc_sh.at[pl.ds(r0, 64)], u_hbm.at[c, pl.ds(r0, 64)])


def _agg16_kernel(y2, ei, zeros16):
    mesh = plsc.VectorSubcoreMesh(core_axis_name="c", subcore_axis_name="s")
    f = pl.kernel(
        _agg16_body,
        out_type=jax.ShapeDtypeStruct((NC, NP, 16), jnp.float32),
        mesh=mesh,
        compiler_params=pltpu.CompilerParams(needs_layout_passes=False, use_tc_tiling_on_sc=False),
        scratch_types=[
            pltpu.VMEM_SHARED((NP, 16), jnp.float32),
            pltpu.VMEM((64,), jnp.int32),
            pltpu.VMEM((64,), jnp.int32),
            pltpu.VMEM((64, 16), jnp.float32),
            pltpu.VMEM((64,), jnp.int32),
            pltpu.VMEM((64,), jnp.int32),
            pltpu.VMEM((64, 16), jnp.float32),
            pltpu.VMEM((64,), jnp.int32),
            pltpu.VMEM((64,), jnp.int32),
            pltpu.VMEM((64, 16), jnp.float32),
            pltpu.VMEM((8,), jnp.int32),
            pltpu.VMEM((8,), jnp.int32),
            pltpu.VMEM((8, 16), jnp.float32),
            pltpu.SemaphoreType.DMA,
            pltpu.SemaphoreType.DMA,
            pltpu.SemaphoreType.DMA,
            pltpu.SemaphoreType.DMA,
            pltpu.SemaphoreType.DMA,
            pltpu.SemaphoreType.DMA,
            pltpu.SemaphoreType.DMA,
        ],
    )
    return f(y2, ei, zeros16)


def _tca_body(d0_ref, d1_ref, x_ref, dinv_ref, y0_ref, y1_ref):
    dinv = lax.rsqrt(d0_ref[...] + d1_ref[...] + 1.0)
    y = x_ref[...] * dinv
    dinv_ref[...] = jnp.broadcast_to(dinv, (BR, 8))
    y0_ref[...] = y[:, :128]
    y1_ref[...] = y[:, 128:]


def _tca(d0, d1, x):
    return pl.pallas_call(
        _tca_body,
        grid=(NP // BR,),
        in_specs=[
            pl.BlockSpec((BR, 1), lambda i: (i, 0)),
            pl.BlockSpec((BR, 1), lambda i: (i, 0)),
            pl.BlockSpec((BR, 256), lambda i: (i, 0)),
        ],
        out_specs=[
            pl.BlockSpec((BR, 8), lambda i: (i, 0)),
            pl.BlockSpec((BR, 128), lambda i: (i, 0)),
            pl.BlockSpec((BR, 128), lambda i: (i, 0)),
        ],
        out_shape=[
            jax.ShapeDtypeStruct((NP, 8), jnp.float32),
            jax.ShapeDtypeStruct((NP, 128), jnp.float32),
            jax.ShapeDtypeStruct((NP, 128), jnp.float32),
        ],
    )(d0, d1, x)


def _tcb_body(t0_ref, t1_ref, y0_ref, y1_ref, dinv_ref, w1_ref, b1_ref,
              w2_ref, y2_ref):
    dinv = dinv_ref[...][:, 0:1]
    a0 = (t0_ref[...] + y0_ref[...]) * dinv
    a1 = (t1_ref[...] + y1_ref[...]) * dinv
    a = jnp.concatenate([a0, a1], axis=1)
    h = jnp.dot(a, w1_ref[...], preferred_element_type=jnp.float32)
    h = jnp.maximum(h + b1_ref[...], 0.0)
    z = jnp.dot(h, w2_ref[...], preferred_element_type=jnp.float32)
    y2_ref[...] = z * dinv


def _tcb(t0, t1, y0, y1, dinv, W1, b1r, W2):
    return pl.pallas_call(
        _tcb_body,
        grid=(NP // BR,),
        in_specs=[
            pl.BlockSpec((BR, 128), lambda i: (i, 0)),
            pl.BlockSpec((BR, 128), lambda i: (i, 0)),
            pl.BlockSpec((BR, 128), lambda i: (i, 0)),
            pl.BlockSpec((BR, 128), lambda i: (i, 0)),
            pl.BlockSpec((BR, 8), lambda i: (i, 0)),
            pl.BlockSpec((256, 512), lambda i: (0, 0)),
            pl.BlockSpec((1, 512), lambda i: (0, 0)),
            pl.BlockSpec((512, 16), lambda i: (0, 0)),
        ],
        out_specs=pl.BlockSpec((BR, 16), lambda i: (i, 0)),
        out_shape=jax.ShapeDtypeStruct((NP, 16), jnp.float32),
    )(t0, t1, y0, y1, dinv, W1, b1r, W2)


def _tcc_body(u_ref, y2_ref, dinv_ref, b2_ref, out_ref):
    u = u_ref[...]
    o = (u[0] + u[1] + y2_ref[...]) * dinv_ref[...][:, 0:1] + b2_ref[...]
    m = jnp.max(o, axis=1, keepdims=True)
    l = o - m
    out_ref[...] = l - jnp.log(jnp.sum(jnp.exp(l), axis=1, keepdims=True))


def _tcc(u, y2, dinv, b2r, n_out):
    return pl.pallas_call(
        _tcc_body,
        grid=(NP // BR,),
        in_specs=[
            pl.BlockSpec((NC, BR, 16), lambda i: (0, i, 0)),
            pl.BlockSpec((BR, 16), lambda i: (i, 0)),
            pl.BlockSpec((BR, 8), lambda i: (i, 0)),
            pl.BlockSpec((1, 16), lambda i: (0, 0)),
        ],
        out_specs=pl.BlockSpec((BR, 16), lambda i: (i, 0)),
        out_shape=jax.ShapeDtypeStruct((n_out, 16), jnp.float32),
    )(u, y2, dinv, b2r)


@jax.jit
def kernel(x, edge_index, W1, b1, W2, b2):
    n = x.shape[0]
    zeros128 = jnp.zeros((128, 128), jnp.float32)
    zeros16 = jnp.zeros((64, 16), jnp.float32)

    deg_c = _deg_kernel(edge_index)
    d0 = deg_c[0].reshape(NP, 1)
    d1 = deg_c[1].reshape(NP, 1)
    dinv, y0, y1 = _tca(d0, d1, x)
    t0, t1 = _agg256_kernel(y0, y1, edge_index, zeros128)
    y2 = _tcb(t0, t1, y0, y1, dinv, W1, b1.reshape(1, -1), W2)
    u = _agg16_kernel(y2, edge_index, zeros16)
    return _tcc(u, y2, dinv, b2.reshape(1, -1), n)


# speedup vs baseline: 18.4110x; 1.0005x over previous
import jax
import jax.numpy as jnp
from jax import lax
from jax.experimental import pallas as pl
from jax.experimental.pallas import tpu as pltpu, tpu_sc as plsc

NP = 10240
BR = 1024
NC = 2
NS = 16


def _zero_1d(ref, nwords):
    z = jnp.zeros((16,), jnp.float32)

    def body(i, _):
        ref[pl.ds(i * 16, 16)] = z
        return 0

    lax.fori_loop(0, nwords // 16, body, 0, unroll=4)


def _deg_body(ei_hbm, deg_hbm, hist_v, idx_v, red_v, shared_h):
    c = lax.axis_index("c")
    s = lax.axis_index("s")
    _zero_1d(hist_v, NP)

    e_tile = 5000
    base = c * (NS * e_tile) + s * e_tile
    pltpu.sync_copy(ei_hbm.at[1, pl.ds(base, e_tile)], idx_v)

    ones = jnp.ones((16,), jnp.float32)

    def body(k, _):
        idx = idx_v[pl.ds(k * 16, 16)]
        plsc.addupdate_scatter(hist_v, [idx], ones)
        return 0

    lax.fori_loop(0, e_tile // 16, body, 0)
    tail_idx = idx_v[pl.ds(e_tile - 16, 16)]
    tail_mask = lax.iota(jnp.int32, 16) >= 8
    plsc.addupdate_scatter(hist_v, [tail_idx], ones, mask=tail_mask)

    pltpu.sync_copy(hist_v, shared_h.at[s])
    plsc.subcore_barrier()
    rows = NP // NS
    for r in range(NS):
        pltpu.sync_copy(shared_h.at[r, pl.ds(s * rows, rows)], red_v.at[r])

    lax.fori_loop(0, rows // 16, _make_sum(red_v, hist_v), 0)
    pltpu.sync_copy(hist_v.at[pl.ds(0, rows)], deg_hbm.at[c, pl.ds(s * rows, rows)])


def _make_sum(red_v, out_v):
    def rbody(k, _):
        acc = red_v[0, pl.ds(k * 16, 16)]
        for r in range(1, NS):
            acc = acc + red_v[r, pl.ds(k * 16, 16)]
        out_v[pl.ds(k * 16, 16)] = acc
        return 0

    return rbody


def _deg_kernel(ei):
    mesh = plsc.VectorSubcoreMesh(core_axis_name="c", subcore_axis_name="s")
    f = pl.kernel(
        _deg_body,
        out_type=jax.ShapeDtypeStruct((NC, NP), jnp.float32),
        mesh=mesh,
        compiler_params=pltpu.CompilerParams(needs_layout_passes=False, use_tc_tiling_on_sc=False),
        scratch_types=[
            pltpu.VMEM((NP,), jnp.float32),
            pltpu.VMEM((5000,), jnp.int32),
            pltpu.VMEM((NS, NP // NS), jnp.float32),
            pltpu.VMEM_SHARED((NS, NP), jnp.float32),
        ],
    )
    return f(ei)


def _edge_pipeline(ei, y_hbm, acc_sh, base, nchunks, bufs):
    assert nchunks % 3 == 0
    CH = bufs[0][2].shape[0]

    def load(b, off):
        pltpu.sync_copy(ei.at[0, pl.ds(off, CH)], b[0])
        pltpu.sync_copy(ei.at[1, pl.ds(off, CH)], b[1])

    def gstart(b):
        pltpu.async_copy(y_hbm.at[b[0]], b[2], b[3])

    def gwait(b):
        pltpu.make_async_copy(y_hbm.at[b[0]], b[2], b[3]).wait()

    def sstart(b):
        pltpu.async_copy(b[2], acc_sh.at[b[1]], b[4], add=True)

    def swait(b):
        pltpu.make_async_copy(b[2], acc_sh.at[b[1]], b[4]).wait()

    load(bufs[0], base)
    gstart(bufs[0])
    load(bufs[1], base + CH)
    gstart(bufs[1])

    def body(k, _):
        for j in range(3):
            b = bufs[j]
            p = bufs[(j + 2) % 3]
            gwait(b)
            sstart(b)
            if j == 0:
                @pl.when(k > 0)
                def _():
                    swait(p)
            else:
                swait(p)
            if j == 0:
                load(p, base + (3 * k + j + 2) * CH)
                gstart(p)
            else:
                @pl.when(3 * k + j + 2 < nchunks)
                def _():
                    load(p, base + (3 * k + j + 2) * CH)
                    gstart(p)
        return 0

    lax.fori_loop(0, nchunks // 3, body, 0)
    swait(bufs[2])


def _agg256_body(y0, y1, ei, zeros_hbm, t0, t1, acc_sh,
                 idx_s0, idx_d0, rows0, idx_s1, idx_d1, rows1,
                 idx_s2, idx_d2, rows2,
                 idx_s_t, idx_d_t, rows_t, gs0, ss0, gs1, ss1, gs2, ss2, sem):
    c = lax.axis_index("c")
    s = lax.axis_index("s")
    rows = NP // NS

    def run(y_hbm, t_hbm):
        for k in range(rows // 128):
            pltpu.sync_copy(zeros_hbm, acc_sh.at[pl.ds(s * rows + k * 128, 128)])
        plsc.subcore_barrier()

        e_tile = 10000
        base = s * e_tile
        _edge_pipeline(ei, y_hbm, acc_sh, base, e_tile // 64,
                       ((idx_s0, idx_d0, rows0, gs0, ss0),
                        (idx_s1, idx_d1, rows1, gs1, ss1),
                        (idx_s2, idx_d2, rows2, gs2, ss2)))
        off = base + (e_tile // 64) * 64
        pltpu.sync_copy(ei.at[0, pl.ds(off, 16)], idx_s_t)
        pltpu.sync_copy(ei.at[1, pl.ds(off, 16)], idx_d_t)
        pltpu.async_copy(y_hbm.at[idx_s_t], rows_t, sem).wait()
        pltpu.sync_copy(rows_t, acc_sh.at[idx_d_t], add=True)

        plsc.subcore_barrier()
        for k in range(rows // 128):
            r0 = s * rows + k * 128
            pltpu.sync_copy(acc_sh.at[pl.ds(r0, 128)], t_hbm.at[pl.ds(r0, 128)])

    @pl.when(c == 0)
    def _():
        run(y0, t0)

    @pl.when(c == 1)
    def _():
        run(y1, t1)


def _agg256_kernel(y0, y1, ei, zeros128):
    mesh = plsc.VectorSubcoreMesh(core_axis_name="c", subcore_axis_name="s")
    f = pl.kernel(
        _agg256_body,
        out_type=[
            jax.ShapeDtypeStruct((NP, 128), jnp.float32),
            jax.ShapeDtypeStruct((NP, 128), jnp.float32),
        ],
        mesh=mesh,
        compiler_params=pltpu.CompilerParams(needs_layout_passes=False, use_tc_tiling_on_sc=False),
        scratch_types=[
            pltpu.VMEM_SHARED((NP, 128), jnp.float32),
            pltpu.VMEM((64,), jnp.int32),
            pltpu.VMEM((64,), jnp.int32),
            pltpu.VMEM((64, 128), jnp.float32),
            pltpu.VMEM((64,), jnp.int32),
            pltpu.VMEM((64,), jnp.int32),
            pltpu.VMEM((64, 128), jnp.float32),
            pltpu.VMEM((64,), jnp.int32),
            pltpu.VMEM((64,), jnp.int32),
            pltpu.VMEM((64, 128), jnp.float32),
            pltpu.VMEM((16,), jnp.int32),
            pltpu.VMEM((16,), jnp.int32),
            pltpu.VMEM((16, 128), jnp.float32),
            pltpu.SemaphoreType.DMA,
            pltpu.SemaphoreType.DMA,
            pltpu.SemaphoreType.DMA,
            pltpu.SemaphoreType.DMA,
            pltpu.SemaphoreType.DMA,
            pltpu.SemaphoreType.DMA,
            pltpu.SemaphoreType.DMA,
        ],
    )
    return f(y0, y1, ei, zeros128)


def _agg16_body(y2, ei, zeros_hbm, u_hbm, acc_sh,
                idx_s0, idx_d0, rows0, idx_s1, idx_d1, rows1,
                idx_s2, idx_d2, rows2,
                idx_s_t, idx_d_t, rows_t, gs0, ss0, gs1, ss1, gs2, ss2, sem):
    c = lax.axis_index("c")
    s = lax.axis_index("s")
    rows = NP // NS

    for k in range(rows // 64):
        pltpu.sync_copy(zeros_hbm, acc_sh.at[pl.ds(s * rows + k * 64, 64)])
    plsc.subcore_barrier()

    e_tile = 5000
    base = c * (NS * e_tile) + s * e_tile
    _edge_pipeline(ei, y2, acc_sh, base, e_tile // 64,
                   ((idx_s0, idx_d0, rows0, gs0, ss0),
                    (idx_s1, idx_d1, rows1, gs1, ss1),
                    (idx_s2, idx_d2, rows2, gs2, ss2)))
    off = base + (e_tile // 64) * 64
    pltpu.sync_copy(ei.at[0, pl.ds(off, 8)], idx_s_t)
    pltpu.sync_copy(ei.at[1, pl.ds(off, 8)], idx_d_t)
    pltpu.async_copy(y2.at[idx_s_t], rows_t, sem).wait()
    pltpu.sync_copy(rows_t, acc_sh.at[idx_d_t], add=True)

    plsc.subcore_barrier()
    for k in range(rows // 64):
        r0 = s * rows + k * 64
        pltpu.sync_copy(acc_sh.at[pl.ds(r0, 64)], u_hbm.at[c, pl.ds(r0, 64)])


def _agg16_kernel(y2, ei, zeros16):
    mesh = plsc.VectorSubcoreMesh(core_axis_name="c", subcore_axis_name="s")
    f = pl.kernel(
        _agg16_body,
        out_type=jax.ShapeDtypeStruct((NC, NP, 16), jnp.float32),
        mesh=mesh,
        compiler_params=pltpu.CompilerParams(needs_layout_passes=False, use_tc_tiling_on_sc=False),
        scratch_types=[
            pltpu.VMEM_SHARED((NP, 16), jnp.float32),
            pltpu.VMEM((64,), jnp.int32),
            pltpu.VMEM((64,), jnp.int32),
            pltpu.VMEM((64, 16), jnp.float32),
            pltpu.VMEM((64,), jnp.int32),
            pltpu.VMEM((64,), jnp.int32),
            pltpu.VMEM((64, 16), jnp.float32),
            pltpu.VMEM((64,), jnp.int32),
            pltpu.VMEM((64,), jnp.int32),
            pltpu.VMEM((64, 16), jnp.float32),
            pltpu.VMEM((8,), jnp.int32),
            pltpu.VMEM((8,), jnp.int32),
            pltpu.VMEM((8, 16), jnp.float32),
            pltpu.SemaphoreType.DMA,
            pltpu.SemaphoreType.DMA,
            pltpu.SemaphoreType.DMA,
            pltpu.SemaphoreType.DMA,
            pltpu.SemaphoreType.DMA,
            pltpu.SemaphoreType.DMA,
            pltpu.SemaphoreType.DMA,
        ],
    )
    return f(y2, ei, zeros16)


def _tca_body(d0_ref, d1_ref, x_ref, dinv_ref, y0_ref, y1_ref):
    dinv = lax.rsqrt(d0_ref[...] + d1_ref[...] + 1.0)
    y = x_ref[...] * dinv
    dinv_ref[...] = jnp.broadcast_to(dinv, (BR, 8))
    y0_ref[...] = y[:, :128]
    y1_ref[...] = y[:, 128:]


def _tca(d0, d1, x):
    return pl.pallas_call(
        _tca_body,
        grid=(NP // BR,),
        in_specs=[
            pl.BlockSpec((BR, 1), lambda i: (i, 0)),
            pl.BlockSpec((BR, 1), lambda i: (i, 0)),
            pl.BlockSpec((BR, 256), lambda i: (i, 0)),
        ],
        out_specs=[
            pl.BlockSpec((BR, 8), lambda i: (i, 0)),
            pl.BlockSpec((BR, 128), lambda i: (i, 0)),
            pl.BlockSpec((BR, 128), lambda i: (i, 0)),
        ],
        out_shape=[
            jax.ShapeDtypeStruct((NP, 8), jnp.float32),
            jax.ShapeDtypeStruct((NP, 128), jnp.float32),
            jax.ShapeDtypeStruct((NP, 128), jnp.float32),
        ],
    )(d0, d1, x)


def _tcb_body(t0_ref, t1_ref, y0_ref, y1_ref, dinv_ref, w1_ref, b1_ref,
              w2_ref, y2_ref):
    dinv = dinv_ref[...][:, 0:1]
    a0 = (t0_ref[...] + y0_ref[...]) * dinv
    a1 = (t1_ref[...] + y1_ref[...]) * dinv
    a = jnp.concatenate([a0, a1], axis=1)
    h = jnp.dot(a, w1_ref[...], preferred_element_type=jnp.float32)
    h = jnp.maximum(h + b1_ref[...], 0.0)
    z = jnp.dot(h, w2_ref[...], preferred_element_type=jnp.float32)
    y2_ref[...] = z * dinv


def _tcb(t0, t1, y0, y1, dinv, W1, b1r, W2):
    return pl.pallas_call(
        _tcb_body,
        grid=(NP // BR,),
        in_specs=[
            pl.BlockSpec((BR, 128), lambda i: (i, 0)),
            pl.BlockSpec((BR, 128), lambda i: (i, 0)),
            pl.BlockSpec((BR, 128), lambda i: (i, 0)),
            pl.BlockSpec((BR, 128), lambda i: (i, 0)),
            pl.BlockSpec((BR, 8), lambda i: (i, 0)),
            pl.BlockSpec((256, 512), lambda i: (0, 0)),
            pl.BlockSpec((1, 512), lambda i: (0, 0)),
            pl.BlockSpec((512, 16), lambda i: (0, 0)),
        ],
        out_specs=pl.BlockSpec((BR, 16), lambda i: (i, 0)),
        out_shape=jax.ShapeDtypeStruct((NP, 16), jnp.float32),
    )(t0, t1, y0, y1, dinv, W1, b1r, W2)


def _tcc_body(u_ref, y2_ref, dinv_ref, b2_ref, out_ref):
    u = u_ref[...]
    o = (u[0] + u[1] + y2_ref[...]) * dinv_ref[...][:, 0:1] + b2_ref[...]
    m = jnp.max(o, axis=1, keepdims=True)
    l = o - m
    out_ref[...] = l - jnp.log(jnp.sum(jnp.exp(l), axis=1, keepdims=True))


def _tcc(u, y2, dinv, b2r, n_out):
    return pl.pallas_call(
        _tcc_body,
        grid=(NP // BR,),
        in_specs=[
            pl.BlockSpec((NC, BR, 16), lambda i: (0, i, 0)),
            pl.BlockSpec((BR, 16), lambda i: (i, 0)),
            pl.BlockSpec((BR, 8), lambda i: (i, 0)),
            pl.BlockSpec((1, 16), lambda i: (0, 0)),
        ],
        out_specs=pl.BlockSpec((BR, 16), lambda i: (i, 0)),
        out_shape=jax.ShapeDtypeStruct((n_out, 16), jnp.float32),
    )(u, y2, dinv, b2r)


@jax.jit
def kernel(x, edge_index, W1, b1, W2, b2):
    n = x.shape[0]
    zeros128 = jnp.zeros((128, 128), jnp.float32)
    zeros16 = jnp.zeros((64, 16), jnp.float32)

    deg_c = _deg_kernel(edge_index)
    d0 = deg_c[0].reshape(NP, 1)
    d1 = deg_c[1].reshape(NP, 1)
    dinv, y0, y1 = _tca(d0, d1, x)
    t0, t1 = _agg256_kernel(y0, y1, edge_index, zeros128)
    y2 = _tcb(t0, t1, y0, y1, dinv, W1, b1.reshape(1, -1), W2)
    u = _agg16_kernel(y2, edge_index, zeros16)
    return _tcc(u, y2, dinv, b2.reshape(1, -1), n)
